# Initial kernel scaffold; baseline (speedup 1.0000x reference)
import jax
import jax.numpy as jnp
from jax.experimental import pallas as pl


def kernel(query_tokens, title_tokens, emb, Wq, bq, Wt, bt):
    raise NotImplementedError("write your pallas kernel here")


# speedup vs baseline: 33.7773x; 33.7773x over previous
import functools

import jax
import jax.numpy as jnp
from jax import lax
from jax.experimental import pallas as pl
from jax.experimental.pallas import tpu as pltpu
from jax.experimental.pallas import tpu_sc as plsc

B, LQ, LT, V, D, H = 16384, 20, 200, 1000, 64, 64
VP = 1024
NC, NS, L = 2, 16, 16
NW = NC * NS
EW = B // NW
CE = 32
NCH = EW // CE
CV = CE * VP

_mesh = plsc.VectorSubcoreMesh(core_axis_name="c", subcore_axis_name="s")


@functools.partial(
    pl.kernel,
    out_type=(
        jax.ShapeDtypeStruct((B * VP,), jnp.float32),
        jax.ShapeDtypeStruct((B * VP,), jnp.float32),
    ),
    mesh=_mesh,
    compiler_params=pltpu.CompilerParams(needs_layout_passes=False),
    scratch_types=(
        pltpu.VMEM((CE * LQ,), jnp.int32),
        pltpu.VMEM((CE * LT,), jnp.int32),
        pltpu.VMEM((CV,), jnp.float32),
        pltpu.VMEM((CV,), jnp.float32),
    ),
)
def _sc_counts(qtok, ttok, cq_hbm, ct_hbm, qtok_v, ttok_v, cq_v, ct_v):
    wid = lax.axis_index("s") * NC + lax.axis_index("c")
    zeros = jnp.zeros((L,), jnp.float32)
    ones = jnp.ones((L,), jnp.float32)
    lane = lax.iota(jnp.int32, L)

    def zinit(i, c):
        cq_v[pl.ds(i * L, L)] = zeros
        ct_v[pl.ds(i * L, L)] = zeros
        return c
    lax.fori_loop(0, CV // L, zinit, 0)

    def scatter_pass(tok_v, cnt_v, npos, add):
        def g_loop(g, c):
            row = g * L + lane
            rowv = row * VP
            rowt = row * npos

            def p_loop(j, c2):
                tok = plsc.load_gather(tok_v, [rowt + j])
                if add:
                    plsc.addupdate_scatter(cnt_v, [rowv + tok], ones)
                else:
                    plsc.store_scatter(cnt_v, [rowv + tok], zeros)
                return c2
            lax.fori_loop(0, npos, p_loop, 0)
            return c
        lax.fori_loop(0, CE // L, g_loop, 0)

    def chunk(k, c):
        e0 = wid * EW + k * CE
        pltpu.sync_copy(qtok.at[pl.ds(e0 * LQ, CE * LQ)], qtok_v)
        pltpu.sync_copy(ttok.at[pl.ds(e0 * LT, CE * LT)], ttok_v)
        scatter_pass(qtok_v, cq_v, LQ, add=True)
        scatter_pass(ttok_v, ct_v, LT, add=True)
        pltpu.sync_copy(cq_v, cq_hbm.at[pl.ds(e0 * VP, CV)])
        pltpu.sync_copy(ct_v, ct_hbm.at[pl.ds(e0 * VP, CV)])
        scatter_pass(qtok_v, cq_v, LQ, add=False)
        scatter_pass(ttok_v, ct_v, LT, add=False)
        return c
    lax.fori_loop(0, NCH, chunk, 0)


BB = 1024


def _tc_body(cq_ref, ct_ref, emb_ref, wq_ref, bq_ref, wt_ref, bt_ref, out_ref):
    f32 = jnp.float32
    qs = jnp.dot(cq_ref[...], emb_ref[...], preferred_element_type=f32)
    ts = jnp.dot(ct_ref[...], emb_ref[...], preferred_element_type=f32)
    qh = jnp.tanh(jnp.dot(qs, wq_ref[...], preferred_element_type=f32) + bq_ref[...])
    th = jnp.tanh(jnp.dot(ts, wt_ref[...], preferred_element_type=f32) + bt_ref[...])
    w12 = jnp.sum(qh * th, axis=1, keepdims=True)
    w1 = jnp.sqrt(jnp.sum(qh * qh, axis=1, keepdims=True))
    w2 = jnp.sqrt(jnp.sum(th * th, axis=1, keepdims=True))
    cos = w12 / (w1 * w2 + 1e-12)
    out_ref[...] = (cos + 1.0) * 0.5


_tc_call = pl.pallas_call(
    _tc_body,
    grid=(B // BB,),
    in_specs=[
        pl.BlockSpec((BB, VP), lambda i: (i, 0)),
        pl.BlockSpec((BB, VP), lambda i: (i, 0)),
        pl.BlockSpec((VP, D), lambda i: (0, 0)),
        pl.BlockSpec((D, H), lambda i: (0, 0)),
        pl.BlockSpec((1, H), lambda i: (0, 0)),
        pl.BlockSpec((D, H), lambda i: (0, 0)),
        pl.BlockSpec((1, H), lambda i: (0, 0)),
    ],
    out_specs=pl.BlockSpec((BB, 1), lambda i: (i, 0)),
    out_shape=jax.ShapeDtypeStruct((B, 1), jnp.float32),
)


def kernel(query_tokens, title_tokens, emb, Wq, bq, Wt, bt):
    qf = query_tokens.reshape(-1).astype(jnp.int32)
    tf = title_tokens.reshape(-1).astype(jnp.int32)
    cq, ct = _sc_counts(qf, tf)
    cq = cq.reshape(B, VP)
    ct = ct.reshape(B, VP)
    emb_pad = jnp.zeros((VP, D), jnp.float32).at[:V].set(emb)
    return _tc_call(cq, ct, emb_pad, Wq.T, bq.reshape(1, H), Wt.T, bt.reshape(1, H))


# speedup vs baseline: 35.6955x; 1.0568x over previous
import functools

import jax
import jax.numpy as jnp
from jax import lax
from jax.experimental import pallas as pl
from jax.experimental.pallas import tpu as pltpu
from jax.experimental.pallas import tpu_sc as plsc

B, LQ, LT, V, D, H = 16384, 20, 200, 1000, 64, 64
VP = 1024
NC, NS, L = 2, 16, 16
NW = NC * NS
EW = B // NW
CE = 32
NCH = EW // CE
CV = CE * VP

_mesh = plsc.VectorSubcoreMesh(core_axis_name="c", subcore_axis_name="s")


@functools.partial(
    pl.kernel,
    out_type=(
        jax.ShapeDtypeStruct((B * VP,), jnp.float32),
        jax.ShapeDtypeStruct((B * VP,), jnp.float32),
    ),
    mesh=_mesh,
    compiler_params=pltpu.CompilerParams(needs_layout_passes=False),
    scratch_types=(
        pltpu.VMEM((CE * LQ + L,), jnp.int32),
        pltpu.VMEM((CE * LT + L,), jnp.int32),
        pltpu.VMEM((CV,), jnp.float32),
        pltpu.VMEM((CV,), jnp.float32),
    ),
)
def _sc_counts(qtok, ttok, cq_hbm, ct_hbm, qtok_v, ttok_v, cq_v, ct_v):
    wid = lax.axis_index("s") * NC + lax.axis_index("c")
    zeros = jnp.zeros((L,), jnp.float32)
    ones = jnp.ones((L,), jnp.float32)
    lane = lax.iota(jnp.int32, L)

    def zinit(i, c):
        cq_v[pl.ds(i * L, L)] = zeros
        ct_v[pl.ds(i * L, L)] = zeros
        return c
    lax.fori_loop(0, CV // L, zinit, 0)

    def scatter_pass(tok_v, cnt_v, npos, add):
        nfull = npos // L
        tail = npos - nfull * L
        tmask = lane < tail

        def upd(idx, mask):
            if add:
                plsc.addupdate_scatter(cnt_v, [idx], ones, mask=mask)
            else:
                plsc.store_scatter(cnt_v, [idx], zeros, mask=mask)

        def e_loop(e, c):
            tbase = e * npos
            rowv = e * VP
            for p in range(nfull):
                tok = tok_v[pl.ds(tbase + p * L, L)]
                upd(rowv + tok, None)
            if tail:
                tok = tok_v[pl.ds(tbase + nfull * L, L)]
                upd(rowv + tok, tmask)
            return c
        lax.fori_loop(0, CE, e_loop, 0)

    def chunk(k, c):
        e0 = wid * EW + k * CE
        pltpu.sync_copy(qtok.at[pl.ds(e0 * LQ, CE * LQ)], qtok_v.at[pl.ds(0, CE * LQ)])
        pltpu.sync_copy(ttok.at[pl.ds(e0 * LT, CE * LT)], ttok_v.at[pl.ds(0, CE * LT)])
        scatter_pass(qtok_v, cq_v, LQ, add=True)
        scatter_pass(ttok_v, ct_v, LT, add=True)
        pltpu.sync_copy(cq_v, cq_hbm.at[pl.ds(e0 * VP, CV)])
        pltpu.sync_copy(ct_v, ct_hbm.at[pl.ds(e0 * VP, CV)])
        scatter_pass(qtok_v, cq_v, LQ, add=False)
        scatter_pass(ttok_v, ct_v, LT, add=False)
        return c
    lax.fori_loop(0, NCH, chunk, 0)


BB = 1024


def _tc_body(cq_ref, ct_ref, emb_ref, wq_ref, bq_ref, wt_ref, bt_ref, out_ref):
    f32 = jnp.float32
    qs = jnp.dot(cq_ref[...], emb_ref[...], preferred_element_type=f32)
    ts = jnp.dot(ct_ref[...], emb_ref[...], preferred_element_type=f32)
    qh = jnp.tanh(jnp.dot(qs, wq_ref[...], preferred_element_type=f32) + bq_ref[...])
    th = jnp.tanh(jnp.dot(ts, wt_ref[...], preferred_element_type=f32) + bt_ref[...])
    w12 = jnp.sum(qh * th, axis=1, keepdims=True)
    w1 = jnp.sqrt(jnp.sum(qh * qh, axis=1, keepdims=True))
    w2 = jnp.sqrt(jnp.sum(th * th, axis=1, keepdims=True))
    cos = w12 / (w1 * w2 + 1e-12)
    out_ref[...] = (cos + 1.0) * 0.5


_tc_call = pl.pallas_call(
    _tc_body,
    grid=(B // BB,),
    in_specs=[
        pl.BlockSpec((BB, VP), lambda i: (i, 0)),
        pl.BlockSpec((BB, VP), lambda i: (i, 0)),
        pl.BlockSpec((VP, D), lambda i: (0, 0)),
        pl.BlockSpec((D, H), lambda i: (0, 0)),
        pl.BlockSpec((1, H), lambda i: (0, 0)),
        pl.BlockSpec((D, H), lambda i: (0, 0)),
        pl.BlockSpec((1, H), lambda i: (0, 0)),
    ],
    out_specs=pl.BlockSpec((BB, 1), lambda i: (i, 0)),
    out_shape=jax.ShapeDtypeStruct((B, 1), jnp.float32),
)


def kernel(query_tokens, title_tokens, emb, Wq, bq, Wt, bt):
    qf = query_tokens.reshape(-1).astype(jnp.int32)
    tf = title_tokens.reshape(-1).astype(jnp.int32)
    cq, ct = _sc_counts(qf, tf)
    cq = cq.reshape(B, VP)
    ct = ct.reshape(B, VP)
    emb_pad = jnp.zeros((VP, D), jnp.float32).at[:V].set(emb)
    return _tc_call(cq, ct, emb_pad, Wq.T, bq.reshape(1, H), Wt.T, bt.reshape(1, H))
